# Initial kernel scaffold; baseline (speedup 1.0000x reference)
import jax
import jax.numpy as jnp
from jax.experimental import pallas as pl


def kernel(edge_attr, emb0, emb1, emb2):
    raise NotImplementedError("write your pallas kernel here")


# speedup vs baseline: 1.8763x; 1.8763x over previous
import functools

import jax
import jax.numpy as jnp
from jax import lax
from jax.experimental import pallas as pl
from jax.experimental.pallas import tpu as pltpu
from jax.experimental.pallas import tpu_sc as plsc

E = 320000
D = 128
NUM_FEAT = 3
NC = 2
NS = 16
NW = NC * NS
BPW = E // NW
C = 80
NCH = BPW // C
LANES = 16
ROW_SLICES = D // LANES

_mesh = plsc.VectorSubcoreMesh(core_axis_name="c", subcore_axis_name="s")


@functools.partial(
    pl.kernel,
    mesh=_mesh,
    out_type=jax.ShapeDtypeStruct((E, D), jnp.float32),
    scratch_types=[
        pltpu.VMEM((NCH, C), jnp.int32),
        pltpu.VMEM((NCH, C), jnp.int32),
        pltpu.VMEM((NCH, C), jnp.int32),
        pltpu.VMEM((C, D), jnp.float32),
        pltpu.VMEM((C, D), jnp.float32),
        pltpu.VMEM((C, D), jnp.float32),
        pltpu.SemaphoreType.DMA,
    ],
)
def _bond_encode(idx_hbm, e0, e1, e2, out, idx0_v, idx1_v, idx2_v,
                 r0, r1, r2, sem):
    cid = lax.axis_index("c")
    sid = lax.axis_index("s")
    wid = sid * NC + cid

    pltpu.sync_copy(idx_hbm.at[0, wid], idx0_v)
    pltpu.sync_copy(idx_hbm.at[1, wid], idx1_v)
    pltpu.sync_copy(idx_hbm.at[2, wid], idx2_v)

    def chunk_body(i, carry):
        cp0 = pltpu.async_copy(e0.at[idx0_v.at[i]], r0, sem)
        cp1 = pltpu.async_copy(e1.at[idx1_v.at[i]], r1, sem)
        cp2 = pltpu.async_copy(e2.at[idx2_v.at[i]], r2, sem)
        cp0.wait()
        cp1.wait()
        cp2.wait()

        def add_row(j, carry2):
            for k in range(ROW_SLICES):
                s = pl.ds(k * LANES, LANES)
                r0[j, s] = r0[j, s] + r1[j, s] + r2[j, s]
            return carry2

        lax.fori_loop(0, C, add_row, 0, unroll=2)

        pltpu.sync_copy(r0, out.at[pl.ds(wid * BPW + i * C, C)])
        return carry

    lax.fori_loop(0, NCH, chunk_body, 0)


def kernel(edge_attr, emb0, emb1, emb2):
    idx = edge_attr.astype(jnp.int32).T.reshape(NUM_FEAT, NW, NCH, C)
    return _bond_encode(idx, emb0, emb1, emb2)
